# baseline (speedup 1.0000x reference)
import jax
import jax.numpy as jnp
from jax.experimental import pallas as pl
from jax.experimental.pallas import tpu as pltpu

B, N, D = 256, 128, 128
H, QD, KD, VD = 4, 32, 32, 32
G = 32


def _leaky(x, alpha=0.2):
    return jnp.where(x >= 0, x, alpha * x)


def _egatv2_kernel(e_ref, x_ref, m_ref, wk_ref, wv_ref, akf_ref, sel_ref,
                   selt_ref, eye_ref, epseye_ref, out_ref):
    x2 = x_ref[...].reshape(G * N, D)
    k_all = _leaky(jnp.dot(x2, wk_ref[...], preferred_element_type=jnp.float32))
    v_all = jnp.dot(x2, wv_ref[...], preferred_element_type=jnp.float32)

    skn = jnp.dot(k_all * akf_ref[...], sel_ref[...],
                  preferred_element_type=jnp.float32)
    esk = jnp.exp(skn)
    et = jnp.dot(esk, selt_ref[...], preferred_element_type=jnp.float32)
    vsc = v_all * et

    eye = eye_ref[...]
    epseye = epseye_ref[...]
    for g in range(G):
        expbase = (jnp.exp(e_ref[g] + epseye)
                   * jnp.maximum(m_ref[g], eye))
        sl = slice(g * N, (g + 1) * N)
        denom = jnp.dot(expbase, esk[sl, :],
                        preferred_element_type=jnp.float32)
        rt = jnp.dot(1.0 / denom, selt_ref[...],
                     preferred_element_type=jnp.float32)
        o = jnp.dot(expbase, vsc[sl, :], preferred_element_type=jnp.float32)
        out_ref[g] = o * rt


def kernel(e, x_atm, m, Wq, Wk, Wv, aq, ak, eps):
    del Wq, aq
    wk_flat = jnp.transpose(Wk, (1, 0, 2)).reshape(D, H * KD)
    wv_flat = jnp.transpose(Wv, (1, 0, 2)).reshape(D, H * VD)
    ak_flat = ak.reshape(1, H * KD)
    sel = (jnp.arange(H * KD)[:, None] // KD ==
           jnp.arange(H)[None, :]).astype(jnp.float32)
    eye = jnp.eye(N, dtype=jnp.float32)
    epseye = eps[0] * eye

    full = lambda shape: pl.BlockSpec(shape, lambda b: (0,) * len(shape))
    per_b = lambda shape: pl.BlockSpec(shape, lambda b: (b, 0, 0))

    return pl.pallas_call(
        _egatv2_kernel,
        grid=(B // G,),
        in_specs=[
            per_b((G, N, N)),
            per_b((G, N, D)),
            per_b((G, N, N)),
            full((D, H * KD)),
            full((D, H * VD)),
            full((1, H * KD)),
            full((H * KD, H)),
            full((H, H * VD)),
            full((N, N)),
            full((N, N)),
        ],
        out_specs=per_b((G, N, H * VD)),
        out_shape=jax.ShapeDtypeStruct((B, N, H * VD), jnp.float32),
        compiler_params=pltpu.CompilerParams(
            dimension_semantics=("parallel",)),
    )(e, x_atm, m, wk_flat, wv_flat, ak_flat, sel, sel.T, eye, epseye)
